# baseline (speedup 1.0000x reference)
import functools

import jax
import jax.numpy as jnp
from jax import lax
from jax.experimental import pallas as pl
from jax.experimental.pallas import tpu as pltpu
from jax.experimental.pallas import tpu_sc as plsc

N = 4096
L = 16
NW = 32
CH = N // NW
NT = N // L
BIG = 3.4e38

_sc_mesh = plsc.VectorSubcoreMesh(core_axis_name="c", subcore_axis_name="s")


@functools.partial(
    pl.kernel,
    out_type=[
        jax.ShapeDtypeStruct((N, L), jnp.float32),
        jax.ShapeDtypeStruct((NW, N), jnp.float32),
    ],
    mesh=_sc_mesh,
    scratch_types=[
        pltpu.VMEM((CH, L), jnp.float32),
        pltpu.VMEM((CH, L), jnp.float32),
        pltpu.VMEM((CH, L), jnp.float32),
        pltpu.VMEM((N,), jnp.float32),
        pltpu.VMEM((N,), jnp.float32),
        pltpu.VMEM((N,), jnp.float32),
        pltpu.VMEM((CH, L), jnp.float32),
        pltpu.VMEM((N,), jnp.float32),
    ],
)
def _sc_stage1(xb0, xb1, xb2, y0, y1, y2, rowv_hbm, colp_hbm,
               xb0v, xb1v, xb2v, y0v, y1v, y2v, rowbuf, colv):
    wid = lax.axis_index("s") * 2 + lax.axis_index("c")
    base = wid * CH

    pltpu.sync_copy(xb0.at[pl.ds(base, CH), :], xb0v)
    pltpu.sync_copy(xb1.at[pl.ds(base, CH), :], xb1v)
    pltpu.sync_copy(xb2.at[pl.ds(base, CH), :], xb2v)
    pltpu.sync_copy(y0, y0v)
    pltpu.sync_copy(y1, y1v)
    pltpu.sync_copy(y2, y2v)

    def _init(t, carry):
        colv[pl.ds(t * L, L)] = jnp.full((L,), BIG, jnp.float32)
        return carry

    lax.fori_loop(0, NT, _init, 0)

    def _row(xi, carry):
        a0 = xb0v[xi, :]
        a1 = xb1v[xi, :]
        a2 = xb2v[xi, :]

        def _ytile(t, rmin):
            s = t * L
            d0 = y0v[pl.ds(s, L)] - a0
            d1 = y1v[pl.ds(s, L)] - a1
            d2 = y2v[pl.ds(s, L)] - a2
            dd = d0 * d0 + d1 * d1 + d2 * d2
            colv[pl.ds(s, L)] = jnp.minimum(colv[pl.ds(s, L)], dd)
            return jnp.minimum(rmin, dd)

        rmin = lax.fori_loop(0, NT, _ytile, jnp.full((L,), BIG, jnp.float32))
        rowbuf[xi, :] = rmin
        return carry

    lax.fori_loop(0, CH, _row, 0)

    pltpu.sync_copy(rowbuf, rowv_hbm.at[pl.ds(base, CH), :])
    pltpu.sync_copy(colv, colp_hbm.at[wid])


def _tc_finish(rowv_ref, colp_ref, out_ref):
    rmin = jnp.min(rowv_ref[...], axis=1)
    cmin = jnp.min(colp_ref[...], axis=0)
    loss = (jnp.sum(jnp.sqrt(rmin)) + jnp.sum(jnp.sqrt(cmin))) * (1.0 / N)
    out_ref[...] = jnp.full((1, 1), loss, dtype=jnp.float32)


def kernel(X, Y):
    Xf = X[0]
    Yf = Y[0]
    xbs = [jnp.broadcast_to(Xf[:, k:k + 1], (N, L)) for k in range(3)]
    ys = [Yf[:, k] for k in range(3)]

    rowv, colp = _sc_stage1(*xbs, *ys)

    out = pl.pallas_call(
        _tc_finish,
        out_shape=jax.ShapeDtypeStruct((1, 1), jnp.float32),
    )(rowv, colp)
    return out[0, 0]


# speedup vs baseline: 1.6739x; 1.6739x over previous
import functools

import jax
import jax.numpy as jnp
from jax import lax
from jax.experimental import pallas as pl
from jax.experimental.pallas import tpu as pltpu
from jax.experimental.pallas import tpu_sc as plsc

N = 4096
L = 16
NW = 32
CH = N // NW
NT = N // L
BX = 8
BIG = 3.4e38

_sc_mesh = plsc.VectorSubcoreMesh(core_axis_name="c", subcore_axis_name="s")


@functools.partial(
    pl.kernel,
    out_type=[
        jax.ShapeDtypeStruct((N, L), jnp.float32),
        jax.ShapeDtypeStruct((NW, N), jnp.float32),
    ],
    mesh=_sc_mesh,
    scratch_types=[
        pltpu.VMEM((CH, L), jnp.float32),
        pltpu.VMEM((CH, L), jnp.float32),
        pltpu.VMEM((CH, L), jnp.float32),
        pltpu.VMEM((N,), jnp.float32),
        pltpu.VMEM((N,), jnp.float32),
        pltpu.VMEM((N,), jnp.float32),
        pltpu.VMEM((CH, L), jnp.float32),
        pltpu.VMEM((N,), jnp.float32),
    ],
)
def _sc_stage1(xb0, xb1, xb2, y0, y1, y2, rowv_hbm, colp_hbm,
               xb0v, xb1v, xb2v, y0v, y1v, y2v, rowbuf, colv):
    wid = lax.axis_index("s") * 2 + lax.axis_index("c")
    base = wid * CH

    pltpu.sync_copy(xb0.at[pl.ds(base, CH), :], xb0v)
    pltpu.sync_copy(xb1.at[pl.ds(base, CH), :], xb1v)
    pltpu.sync_copy(xb2.at[pl.ds(base, CH), :], xb2v)
    pltpu.sync_copy(y0, y0v)
    pltpu.sync_copy(y1, y1v)
    pltpu.sync_copy(y2, y2v)

    def _init(t, carry):
        colv[pl.ds(t * L, L)] = jnp.full((L,), BIG, jnp.float32)
        return carry

    lax.fori_loop(0, NT, _init, 0)

    def _chunk(c, carry):
        xi0 = c * BX
        a0 = [xb0v[xi0 + k, :] for k in range(BX)]
        a1 = [xb1v[xi0 + k, :] for k in range(BX)]
        a2 = [xb2v[xi0 + k, :] for k in range(BX)]

        def _ytile(t, rmins):
            s = t * L
            b0 = y0v[pl.ds(s, L)]
            b1 = y1v[pl.ds(s, L)]
            b2 = y2v[pl.ds(s, L)]
            dds = []
            for k in range(BX):
                d0 = b0 - a0[k]
                d1 = b1 - a1[k]
                d2 = b2 - a2[k]
                dds.append(d0 * d0 + d1 * d1 + d2 * d2)
            cc = dds[0]
            for k in range(1, BX):
                cc = jnp.minimum(cc, dds[k])
            colv[pl.ds(s, L)] = jnp.minimum(colv[pl.ds(s, L)], cc)
            return tuple(jnp.minimum(r, d) for r, d in zip(rmins, dds))

        init = tuple(jnp.full((L,), BIG, jnp.float32) for _ in range(BX))
        rmins = lax.fori_loop(0, NT, _ytile, init)
        for k in range(BX):
            rowbuf[xi0 + k, :] = rmins[k]
        return carry

    lax.fori_loop(0, CH // BX, _chunk, 0)

    pltpu.sync_copy(rowbuf, rowv_hbm.at[pl.ds(base, CH), :])
    pltpu.sync_copy(colv, colp_hbm.at[wid])


def _tc_finish(rowv_ref, colp_ref, out_ref):
    rmin = jnp.min(rowv_ref[...], axis=1)
    cmin = jnp.min(colp_ref[...], axis=0)
    loss = (jnp.sum(jnp.sqrt(rmin)) + jnp.sum(jnp.sqrt(cmin))) * (1.0 / N)
    out_ref[...] = jnp.full((1, 1), loss, dtype=jnp.float32)


def kernel(X, Y):
    Xf = X[0]
    Yf = Y[0]
    xbs = [jnp.broadcast_to(Xf[:, k:k + 1], (N, L)) for k in range(3)]
    ys = [Yf[:, k] for k in range(3)]

    rowv, colp = _sc_stage1(*xbs, *ys)

    out = pl.pallas_call(
        _tc_finish,
        out_shape=jax.ShapeDtypeStruct((1, 1), jnp.float32),
    )(rowv, colp)
    return out[0, 0]


# speedup vs baseline: 1.7807x; 1.0638x over previous
import functools

import jax
import jax.numpy as jnp
from jax import lax
from jax.experimental import pallas as pl
from jax.experimental.pallas import tpu as pltpu
from jax.experimental.pallas import tpu_sc as plsc

N = 4096
L = 16
NW = 32
CH = N // NW
NT = N // L
BX = 8
BIG = 3.4e38

_sc_mesh = plsc.VectorSubcoreMesh(core_axis_name="c", subcore_axis_name="s")


@functools.partial(
    pl.kernel,
    out_type=[
        jax.ShapeDtypeStruct((N, L), jnp.float32),
        jax.ShapeDtypeStruct((NW, N), jnp.float32),
    ],
    mesh=_sc_mesh,
    scratch_types=[
        pltpu.VMEM((CH, L), jnp.float32),
        pltpu.VMEM((CH, L), jnp.float32),
        pltpu.VMEM((CH, L), jnp.float32),
        pltpu.VMEM((N,), jnp.float32),
        pltpu.VMEM((N,), jnp.float32),
        pltpu.VMEM((N,), jnp.float32),
        pltpu.VMEM((N,), jnp.float32),
        pltpu.VMEM((CH, L), jnp.float32),
        pltpu.VMEM((N,), jnp.float32),
    ],
)
def _sc_stage1(xb0, xb1, xb2, ym0, ym1, ym2, yn, rowv_hbm, colp_hbm,
               xb0v, xb1v, xb2v, ym0v, ym1v, ym2v, ynv, rowbuf, colv):
    wid = lax.axis_index("s") * 2 + lax.axis_index("c")
    base = wid * CH

    pltpu.sync_copy(xb0.at[pl.ds(base, CH), :], xb0v)
    pltpu.sync_copy(xb1.at[pl.ds(base, CH), :], xb1v)
    pltpu.sync_copy(xb2.at[pl.ds(base, CH), :], xb2v)
    pltpu.sync_copy(ym0, ym0v)
    pltpu.sync_copy(ym1, ym1v)
    pltpu.sync_copy(ym2, ym2v)
    pltpu.sync_copy(yn, ynv)

    def _init(t, carry):
        colv[pl.ds(t * L, L)] = jnp.full((L,), BIG, jnp.float32)
        return carry

    lax.fori_loop(0, NT, _init, 0)

    def _chunk(c, carry):
        xi0 = c * BX
        a0 = [xb0v[xi0 + k, :] for k in range(BX)]
        a1 = [xb1v[xi0 + k, :] for k in range(BX)]
        a2 = [xb2v[xi0 + k, :] for k in range(BX)]
        an = [a0[k] * a0[k] + a1[k] * a1[k] + a2[k] * a2[k]
              for k in range(BX)]

        def _ytile(t, rmins):
            s = t * L
            b0 = ym0v[pl.ds(s, L)]
            b1 = ym1v[pl.ds(s, L)]
            b2 = ym2v[pl.ds(s, L)]
            bn = ynv[pl.ds(s, L)]
            tts = []
            for k in range(BX):
                tt = a0[k] * b0 + bn
                tt = tt + a1[k] * b1
                tt = tt + a2[k] * b2
                tts.append(tt)
            cc = tts[0] + an[0]
            for k in range(1, BX):
                cc = jnp.minimum(cc, tts[k] + an[k])
            colv[pl.ds(s, L)] = jnp.minimum(colv[pl.ds(s, L)], cc)
            return tuple(jnp.minimum(r, t_) for r, t_ in zip(rmins, tts))

        init = tuple(jnp.full((L,), BIG, jnp.float32) for _ in range(BX))
        rmins = lax.fori_loop(0, NT, _ytile, init)
        for k in range(BX):
            rowbuf[xi0 + k, :] = rmins[k] + an[k]
        return carry

    lax.fori_loop(0, CH // BX, _chunk, 0)

    pltpu.sync_copy(rowbuf, rowv_hbm.at[pl.ds(base, CH), :])
    pltpu.sync_copy(colv, colp_hbm.at[wid])


def _tc_finish(rowv_ref, colp_ref, out_ref):
    rmin = jnp.min(rowv_ref[...], axis=1)
    cmin = jnp.min(colp_ref[...], axis=0)
    rmin = jnp.maximum(rmin, 0.0)
    cmin = jnp.maximum(cmin, 0.0)
    loss = (jnp.sum(jnp.sqrt(rmin)) + jnp.sum(jnp.sqrt(cmin))) * (1.0 / N)
    out_ref[...] = jnp.full((1, 1), loss, dtype=jnp.float32)


def kernel(X, Y):
    Xf = X[0]
    Yf = Y[0]
    xbs = [jnp.broadcast_to(Xf[:, k:k + 1], (N, L)) for k in range(3)]
    yms = [-2.0 * Yf[:, k] for k in range(3)]
    yn = Yf[:, 0] ** 2 + Yf[:, 1] ** 2 + Yf[:, 2] ** 2

    rowv, colp = _sc_stage1(*xbs, *yms, yn)

    out = pl.pallas_call(
        _tc_finish,
        out_shape=jax.ShapeDtypeStruct((1, 1), jnp.float32),
    )(rowv, colp)
    return out[0, 0]
